# Initial kernel scaffold; baseline (speedup 1.0000x reference)
import jax
import jax.numpy as jnp
from jax.experimental import pallas as pl


def kernel(x, edge_index, W1_l, W1_r, b1, W2_l, W2_r, b2):
    raise NotImplementedError("write your pallas kernel here")


# speedup vs baseline: 11.2052x; 11.2052x over previous
import functools

import jax
import jax.numpy as jnp
from jax import lax
from jax.experimental import pallas as pl
from jax.experimental.pallas import tpu as pltpu
from jax.experimental.pallas import tpu_sc as plsc

NC = 2
NS = 16
NW = NC * NS
CHUNK = 100
CW = 16

_MESH = plsc.VectorSubcoreMesh(
    core_axis_name="c", subcore_axis_name="s", num_cores=NC, num_subcores=NS
)


def _make_sc_edge_pass(n, h, n_chunks, do_cnt):
  step = ((n // NS) // 8) * 8
  stripe_rows = n - (NS - 1) * step
  out_type = [jax.ShapeDtypeStruct((NC, n, h), jnp.float32)]
  scratch = [
      pltpu.VMEM((n_chunks, CHUNK), jnp.int32),
      pltpu.VMEM((n_chunks, CHUNK), jnp.int32),
      pltpu.VMEM((CHUNK, h), jnp.float32),
      pltpu.VMEM_SHARED((n, h), jnp.float32),
      pltpu.SemaphoreType.DMA,
  ]
  if do_cnt:
    out_type.append(jax.ShapeDtypeStruct((NC, n, CW), jnp.float32))
    scratch += [
        pltpu.VMEM((CHUNK, CW), jnp.float32),
        pltpu.VMEM_SHARED((n, CW), jnp.float32),
    ]

  def body(*refs):
    if do_cnt:
      (p_hbm, z32, z16, ones_hbm, src_hbm, dst_hbm,
       acc_out, cnt_out, srcv, dstv, rows, acc_sp, sem, onesv, cnt_sp) = refs
    else:
      (p_hbm, z32, src_hbm, dst_hbm,
       acc_out, srcv, dstv, rows, acc_sp, sem) = refs
    c = lax.axis_index("c")
    s = lax.axis_index("s")
    wid = s * NC + c
    r0 = pl.multiple_of(s * step, 8)
    stripe = pl.ds(r0, stripe_rows)
    pltpu.sync_copy(z32, acc_sp.at[stripe])
    if do_cnt:
      pltpu.sync_copy(z16, cnt_sp.at[stripe])
      pltpu.sync_copy(ones_hbm, onesv)
    pltpu.sync_copy(src_hbm.at[wid], srcv)
    pltpu.sync_copy(dst_hbm.at[wid], dstv)
    plsc.subcore_barrier()

    def edge_step(j, carry):
      pltpu.async_copy(p_hbm.at[srcv.at[j]], rows, sem).wait()
      pltpu.sync_copy(rows, acc_sp.at[dstv.at[j]], add=True)
      if do_cnt:
        pltpu.sync_copy(onesv, cnt_sp.at[dstv.at[j]], add=True)
      return carry

    lax.fori_loop(0, n_chunks, edge_step, 0)
    plsc.subcore_barrier()
    pltpu.sync_copy(acc_sp.at[stripe], acc_out.at[c, stripe])
    if do_cnt:
      pltpu.sync_copy(cnt_sp.at[stripe], cnt_out.at[c, stripe])

  return pl.kernel(body, out_type=out_type if do_cnt else out_type[0],
                   mesh=_MESH, scratch_types=scratch,
                   compiler_params=pltpu.CompilerParams(
                       use_tc_tiling_on_sc=False))


def _mm_dual(x, wl, wr, block_rows):
  n, d = x.shape
  h = wl.shape[1]

  def body(x_ref, wl_ref, wr_ref, p_ref, r_ref):
    xb = x_ref[...]
    p_ref[...] = jnp.dot(xb, wl_ref[...], preferred_element_type=jnp.float32)
    r_ref[...] = jnp.dot(xb, wr_ref[...], preferred_element_type=jnp.float32)

  return pl.pallas_call(
      body,
      grid=(n // block_rows,),
      in_specs=[
          pl.BlockSpec((block_rows, d), lambda i: (i, 0)),
          pl.BlockSpec((d, h), lambda i: (0, 0)),
          pl.BlockSpec((d, h), lambda i: (0, 0)),
      ],
      out_specs=[
          pl.BlockSpec((block_rows, h), lambda i: (i, 0)),
          pl.BlockSpec((block_rows, h), lambda i: (i, 0)),
      ],
      out_shape=[
          jax.ShapeDtypeStruct((n, h), jnp.float32),
          jax.ShapeDtypeStruct((n, h), jnp.float32),
      ],
  )(x, wl, wr)


def _combine_mm(acc, cnt, r1, b1, w2l, w2r, block_rows):
  _, n, h = acc.shape
  h2 = w2l.shape[1]

  def body(acc_ref, cnt_ref, r1_ref, b1_ref, w2l_ref, w2r_ref, p_ref, r_ref):
    a = acc_ref[0] + acc_ref[1]
    cn = jnp.maximum(cnt_ref[0, :, 0:1] + cnt_ref[1, :, 0:1], 1.0)
    hb = jnp.maximum(a / cn + r1_ref[...] + b1_ref[...], 0.0)
    p_ref[...] = jnp.dot(hb, w2l_ref[...], preferred_element_type=jnp.float32)
    r_ref[...] = jnp.dot(hb, w2r_ref[...], preferred_element_type=jnp.float32)

  return pl.pallas_call(
      body,
      grid=(n // block_rows,),
      in_specs=[
          pl.BlockSpec((NC, block_rows, h), lambda i: (0, i, 0)),
          pl.BlockSpec((NC, block_rows, CW), lambda i: (0, i, 0)),
          pl.BlockSpec((block_rows, h), lambda i: (i, 0)),
          pl.BlockSpec((1, h), lambda i: (0, 0)),
          pl.BlockSpec((h, h2), lambda i: (0, 0)),
          pl.BlockSpec((h, h2), lambda i: (0, 0)),
      ],
      out_specs=[
          pl.BlockSpec((block_rows, h2), lambda i: (i, 0)),
          pl.BlockSpec((block_rows, h2), lambda i: (i, 0)),
      ],
      out_shape=[
          jax.ShapeDtypeStruct((n, h2), jnp.float32),
          jax.ShapeDtypeStruct((n, h2), jnp.float32),
      ],
  )(acc, cnt, r1, b1, w2l, w2r)


def _combine_final(acc, cnt, r2, b2, block_rows):
  _, n, h = acc.shape

  def body(acc_ref, cnt_ref, r2_ref, b2_ref, o_ref):
    a = acc_ref[0] + acc_ref[1]
    cn = jnp.maximum(cnt_ref[0, :, 0:1] + cnt_ref[1, :, 0:1], 1.0)
    o_ref[...] = jnp.maximum(a / cn + r2_ref[...] + b2_ref[...], 0.0)

  return pl.pallas_call(
      body,
      grid=(n // block_rows,),
      in_specs=[
          pl.BlockSpec((NC, block_rows, h), lambda i: (0, i, 0)),
          pl.BlockSpec((NC, block_rows, CW), lambda i: (0, i, 0)),
          pl.BlockSpec((block_rows, h), lambda i: (i, 0)),
          pl.BlockSpec((1, h), lambda i: (0, 0)),
      ],
      out_specs=pl.BlockSpec((block_rows, h), lambda i: (i, 0)),
      out_shape=jax.ShapeDtypeStruct((n, h), jnp.float32),
  )(acc, cnt, r2, b2)


def kernel(x, edge_index, W1_l, W1_r, b1, W2_l, W2_r, b2):
  n, d = x.shape
  h = W1_l.shape[1]
  e = edge_index.shape[1]
  n_chunks = e // (NW * CHUNK)
  block_rows = 1000

  step = ((n // NS) // 8) * 8
  stripe_rows = n - (NS - 1) * step
  src2 = edge_index[0].reshape(NW, n_chunks, CHUNK)
  dst2 = edge_index[1].reshape(NW, n_chunks, CHUNK)
  z32 = jnp.zeros((stripe_rows, h), jnp.float32)
  z16 = jnp.zeros((stripe_rows, CW), jnp.float32)
  ones = jnp.ones((CHUNK, CW), jnp.float32)

  sc_pass1 = _make_sc_edge_pass(n, h, n_chunks, do_cnt=True)
  sc_pass2 = _make_sc_edge_pass(n, h, n_chunks, do_cnt=False)

  p1, r1 = _mm_dual(x, W1_l, W1_r, block_rows)
  acc1, cnt = sc_pass1(p1, z32, z16, ones, src2, dst2)
  p2, r2 = _combine_mm(acc1, cnt, r1, b1.reshape(1, h), W2_l, W2_r, block_rows)
  acc2 = sc_pass2(p2, z32, src2, dst2)
  return _combine_final(acc2, cnt, r2, b2.reshape(1, h), block_rows)


# speedup vs baseline: 15.1180x; 1.3492x over previous
import functools

import jax
import jax.numpy as jnp
from jax import lax
from jax.experimental import pallas as pl
from jax.experimental.pallas import tpu as pltpu
from jax.experimental.pallas import tpu_sc as plsc

NC = 2
NS = 16
NW = NC * NS
CHUNK = 100
CW = 16

_MESH = plsc.VectorSubcoreMesh(
    core_axis_name="c", subcore_axis_name="s", num_cores=NC, num_subcores=NS
)


def _make_sc_edge_pass(n, h, n_chunks, do_cnt):
  step = ((n // NS) // 8) * 8
  stripe_rows = n - (NS - 1) * step
  out_type = [jax.ShapeDtypeStruct((NC, n, h), jnp.float32)]
  scratch = [
      pltpu.VMEM((n_chunks, CHUNK), jnp.int32),
      pltpu.VMEM((n_chunks, CHUNK), jnp.int32),
      pltpu.VMEM((CHUNK, h), jnp.float32),
      pltpu.VMEM_SHARED((n, h), jnp.float32),
      pltpu.VMEM_SHARED((n, h), jnp.float32),
      pltpu.SemaphoreType.DMA,
  ]
  if do_cnt:
    out_type.append(jax.ShapeDtypeStruct((NC, n, CW), jnp.float32))
    scratch += [
        pltpu.VMEM((CHUNK, CW), jnp.float32),
        pltpu.VMEM_SHARED((n, CW), jnp.float32),
    ]

  def body(*refs):
    if do_cnt:
      (p_hbm, z32, z16, ones_hbm, src_hbm, dst_hbm,
       acc_out, cnt_out, srcv, dstv, rows, acc_sp, tbl_sp, sem,
       onesv, cnt_sp) = refs
    else:
      (p_hbm, z32, src_hbm, dst_hbm,
       acc_out, srcv, dstv, rows, acc_sp, tbl_sp, sem) = refs
    c = lax.axis_index("c")
    s = lax.axis_index("s")
    wid = s * NC + c
    r0 = pl.multiple_of(s * step, 8)
    stripe = pl.ds(r0, stripe_rows)
    pltpu.sync_copy(p_hbm.at[stripe], tbl_sp.at[stripe])
    pltpu.sync_copy(z32, acc_sp.at[stripe])
    if do_cnt:
      pltpu.sync_copy(z16, cnt_sp.at[stripe])
      pltpu.sync_copy(ones_hbm, onesv)
    pltpu.sync_copy(src_hbm.at[wid], srcv)
    pltpu.sync_copy(dst_hbm.at[wid], dstv)
    plsc.subcore_barrier()

    def edge_step(j, carry):
      pltpu.async_copy(tbl_sp.at[srcv.at[j]], rows, sem).wait()
      pltpu.sync_copy(rows, acc_sp.at[dstv.at[j]], add=True)
      if do_cnt:
        pltpu.sync_copy(onesv, cnt_sp.at[dstv.at[j]], add=True)
      return carry

    lax.fori_loop(0, n_chunks, edge_step, 0)
    plsc.subcore_barrier()
    pltpu.sync_copy(acc_sp.at[stripe], acc_out.at[c, stripe])
    if do_cnt:
      pltpu.sync_copy(cnt_sp.at[stripe], cnt_out.at[c, stripe])

  return pl.kernel(body, out_type=out_type if do_cnt else out_type[0],
                   mesh=_MESH, scratch_types=scratch,
                   compiler_params=pltpu.CompilerParams(
                       use_tc_tiling_on_sc=False))


def _mm_dual(x, wl, wr, block_rows):
  n, d = x.shape
  h = wl.shape[1]

  def body(x_ref, wl_ref, wr_ref, p_ref, r_ref):
    xb = x_ref[...]
    p_ref[...] = jnp.dot(xb, wl_ref[...], preferred_element_type=jnp.float32)
    r_ref[...] = jnp.dot(xb, wr_ref[...], preferred_element_type=jnp.float32)

  return pl.pallas_call(
      body,
      grid=(n // block_rows,),
      in_specs=[
          pl.BlockSpec((block_rows, d), lambda i: (i, 0)),
          pl.BlockSpec((d, h), lambda i: (0, 0)),
          pl.BlockSpec((d, h), lambda i: (0, 0)),
      ],
      out_specs=[
          pl.BlockSpec((block_rows, h), lambda i: (i, 0)),
          pl.BlockSpec((block_rows, h), lambda i: (i, 0)),
      ],
      out_shape=[
          jax.ShapeDtypeStruct((n, h), jnp.float32),
          jax.ShapeDtypeStruct((n, h), jnp.float32),
      ],
  )(x, wl, wr)


def _combine_mm(acc, cnt, r1, b1, w2l, w2r, block_rows):
  _, n, h = acc.shape
  h2 = w2l.shape[1]

  def body(acc_ref, cnt_ref, r1_ref, b1_ref, w2l_ref, w2r_ref, p_ref, r_ref):
    a = acc_ref[0] + acc_ref[1]
    cn = jnp.maximum(cnt_ref[0, :, 0:1] + cnt_ref[1, :, 0:1], 1.0)
    hb = jnp.maximum(a / cn + r1_ref[...] + b1_ref[...], 0.0)
    p_ref[...] = jnp.dot(hb, w2l_ref[...], preferred_element_type=jnp.float32)
    r_ref[...] = jnp.dot(hb, w2r_ref[...], preferred_element_type=jnp.float32)

  return pl.pallas_call(
      body,
      grid=(n // block_rows,),
      in_specs=[
          pl.BlockSpec((NC, block_rows, h), lambda i: (0, i, 0)),
          pl.BlockSpec((NC, block_rows, CW), lambda i: (0, i, 0)),
          pl.BlockSpec((block_rows, h), lambda i: (i, 0)),
          pl.BlockSpec((1, h), lambda i: (0, 0)),
          pl.BlockSpec((h, h2), lambda i: (0, 0)),
          pl.BlockSpec((h, h2), lambda i: (0, 0)),
      ],
      out_specs=[
          pl.BlockSpec((block_rows, h2), lambda i: (i, 0)),
          pl.BlockSpec((block_rows, h2), lambda i: (i, 0)),
      ],
      out_shape=[
          jax.ShapeDtypeStruct((n, h2), jnp.float32),
          jax.ShapeDtypeStruct((n, h2), jnp.float32),
      ],
  )(acc, cnt, r1, b1, w2l, w2r)


def _combine_final(acc, cnt, r2, b2, block_rows):
  _, n, h = acc.shape

  def body(acc_ref, cnt_ref, r2_ref, b2_ref, o_ref):
    a = acc_ref[0] + acc_ref[1]
    cn = jnp.maximum(cnt_ref[0, :, 0:1] + cnt_ref[1, :, 0:1], 1.0)
    o_ref[...] = jnp.maximum(a / cn + r2_ref[...] + b2_ref[...], 0.0)

  return pl.pallas_call(
      body,
      grid=(n // block_rows,),
      in_specs=[
          pl.BlockSpec((NC, block_rows, h), lambda i: (0, i, 0)),
          pl.BlockSpec((NC, block_rows, CW), lambda i: (0, i, 0)),
          pl.BlockSpec((block_rows, h), lambda i: (i, 0)),
          pl.BlockSpec((1, h), lambda i: (0, 0)),
      ],
      out_specs=pl.BlockSpec((block_rows, h), lambda i: (i, 0)),
      out_shape=jax.ShapeDtypeStruct((n, h), jnp.float32),
  )(acc, cnt, r2, b2)


def kernel(x, edge_index, W1_l, W1_r, b1, W2_l, W2_r, b2):
  n, d = x.shape
  h = W1_l.shape[1]
  e = edge_index.shape[1]
  n_chunks = e // (NW * CHUNK)
  block_rows = 1000

  step = ((n // NS) // 8) * 8
  stripe_rows = n - (NS - 1) * step
  src2 = edge_index[0].reshape(NW, n_chunks, CHUNK)
  dst2 = edge_index[1].reshape(NW, n_chunks, CHUNK)
  z32 = jnp.zeros((stripe_rows, h), jnp.float32)
  z16 = jnp.zeros((stripe_rows, CW), jnp.float32)
  ones = jnp.ones((CHUNK, CW), jnp.float32)

  sc_pass1 = _make_sc_edge_pass(n, h, n_chunks, do_cnt=True)
  sc_pass2 = _make_sc_edge_pass(n, h, n_chunks, do_cnt=False)

  p1, r1 = _mm_dual(x, W1_l, W1_r, block_rows)
  acc1, cnt = sc_pass1(p1, z32, z16, ones, src2, dst2)
  p2, r2 = _combine_mm(acc1, cnt, r1, b1.reshape(1, h), W2_l, W2_r, block_rows)
  acc2 = sc_pass2(p2, z32, src2, dst2)
  return _combine_final(acc2, cnt, r2, b2.reshape(1, h), block_rows)


# speedup vs baseline: 17.4961x; 1.1573x over previous
import functools

import jax
import jax.numpy as jnp
from jax import lax
from jax.experimental import pallas as pl
from jax.experimental.pallas import tpu as pltpu
from jax.experimental.pallas import tpu_sc as plsc

NC = 2
NS = 16
NW = NC * NS
CHUNK = 100
CW = 16

_MESH = plsc.VectorSubcoreMesh(
    core_axis_name="c", subcore_axis_name="s", num_cores=NC, num_subcores=NS
)


def _make_sc_edge_pass(n, h, n_chunks, do_cnt):
  step = ((n // NS) // 8) * 8
  stripe_rows = n - (NS - 1) * step
  out_type = [jax.ShapeDtypeStruct((NC, n, h), jnp.float32)]
  scratch = [
      pltpu.VMEM((n_chunks, CHUNK), jnp.int32),
      pltpu.VMEM((n_chunks, CHUNK), jnp.int32),
      pltpu.VMEM((CHUNK, h), jnp.float32),
      pltpu.VMEM((CHUNK, h), jnp.float32),
      pltpu.VMEM_SHARED((n, h), jnp.float32),
      pltpu.VMEM_SHARED((n, h), jnp.float32),
      pltpu.SemaphoreType.DMA,
      pltpu.SemaphoreType.DMA,
  ]
  if do_cnt:
    out_type.append(jax.ShapeDtypeStruct((NC, n, CW), jnp.float32))
    scratch += [
        pltpu.VMEM((CHUNK, CW), jnp.float32),
        pltpu.VMEM_SHARED((n, CW), jnp.float32),
    ]

  def body(*refs):
    if do_cnt:
      (p_hbm, z32, z16, ones_hbm, src_hbm, dst_hbm,
       acc_out, cnt_out, srcv, dstv, rows0, rows1, acc_sp, tbl_sp,
       sem0, sem1, onesv, cnt_sp) = refs
    else:
      (p_hbm, z32, src_hbm, dst_hbm,
       acc_out, srcv, dstv, rows0, rows1, acc_sp, tbl_sp, sem0, sem1) = refs
    c = lax.axis_index("c")
    s = lax.axis_index("s")
    wid = s * NC + c
    r0 = pl.multiple_of(s * step, 8)
    stripe = pl.ds(r0, stripe_rows)
    pltpu.sync_copy(p_hbm.at[stripe], tbl_sp.at[stripe])
    pltpu.sync_copy(z32, acc_sp.at[stripe])
    if do_cnt:
      pltpu.sync_copy(z16, cnt_sp.at[stripe])
      pltpu.sync_copy(ones_hbm, onesv)
    pltpu.sync_copy(src_hbm.at[wid], srcv)
    pltpu.sync_copy(dst_hbm.at[wid], dstv)
    plsc.subcore_barrier()

    def scatter(buf, j):
      pltpu.sync_copy(buf, acc_sp.at[dstv.at[j]], add=True)
      if do_cnt:
        pltpu.sync_copy(onesv, cnt_sp.at[dstv.at[j]], add=True)

    pltpu.async_copy(tbl_sp.at[srcv.at[0]], rows0, sem0)

    def edge_step(i, carry):
      j0 = 2 * i
      pltpu.async_copy(tbl_sp.at[srcv.at[j0 + 1]], rows1, sem1)
      pltpu.make_async_copy(tbl_sp.at[srcv.at[j0]], rows0, sem0).wait()
      scatter(rows0, j0)

      @pl.when(j0 + 2 < n_chunks)
      def _():
        pltpu.async_copy(tbl_sp.at[srcv.at[j0 + 2]], rows0, sem0)

      pltpu.make_async_copy(tbl_sp.at[srcv.at[j0 + 1]], rows1, sem1).wait()
      scatter(rows1, j0 + 1)
      return carry

    lax.fori_loop(0, n_chunks // 2, edge_step, 0)
    plsc.subcore_barrier()
    pltpu.sync_copy(acc_sp.at[stripe], acc_out.at[c, stripe])
    if do_cnt:
      pltpu.sync_copy(cnt_sp.at[stripe], cnt_out.at[c, stripe])

  return pl.kernel(body, out_type=out_type if do_cnt else out_type[0],
                   mesh=_MESH, scratch_types=scratch,
                   compiler_params=pltpu.CompilerParams(
                       use_tc_tiling_on_sc=False))


def _mm_dual(x, wl, wr, block_rows):
  n, d = x.shape
  h = wl.shape[1]

  def body(x_ref, wl_ref, wr_ref, p_ref, r_ref):
    xb = x_ref[...]
    p_ref[...] = jnp.dot(xb, wl_ref[...], preferred_element_type=jnp.float32)
    r_ref[...] = jnp.dot(xb, wr_ref[...], preferred_element_type=jnp.float32)

  return pl.pallas_call(
      body,
      grid=(n // block_rows,),
      in_specs=[
          pl.BlockSpec((block_rows, d), lambda i: (i, 0)),
          pl.BlockSpec((d, h), lambda i: (0, 0)),
          pl.BlockSpec((d, h), lambda i: (0, 0)),
      ],
      out_specs=[
          pl.BlockSpec((block_rows, h), lambda i: (i, 0)),
          pl.BlockSpec((block_rows, h), lambda i: (i, 0)),
      ],
      out_shape=[
          jax.ShapeDtypeStruct((n, h), jnp.float32),
          jax.ShapeDtypeStruct((n, h), jnp.float32),
      ],
  )(x, wl, wr)


def _combine_mm(acc, cnt, r1, b1, w2l, w2r, block_rows):
  _, n, h = acc.shape
  h2 = w2l.shape[1]

  def body(acc_ref, cnt_ref, r1_ref, b1_ref, w2l_ref, w2r_ref, p_ref, r_ref):
    a = acc_ref[0] + acc_ref[1]
    cn = jnp.maximum(cnt_ref[0, :, 0:1] + cnt_ref[1, :, 0:1], 1.0)
    hb = jnp.maximum(a / cn + r1_ref[...] + b1_ref[...], 0.0)
    p_ref[...] = jnp.dot(hb, w2l_ref[...], preferred_element_type=jnp.float32)
    r_ref[...] = jnp.dot(hb, w2r_ref[...], preferred_element_type=jnp.float32)

  return pl.pallas_call(
      body,
      grid=(n // block_rows,),
      in_specs=[
          pl.BlockSpec((NC, block_rows, h), lambda i: (0, i, 0)),
          pl.BlockSpec((NC, block_rows, CW), lambda i: (0, i, 0)),
          pl.BlockSpec((block_rows, h), lambda i: (i, 0)),
          pl.BlockSpec((1, h), lambda i: (0, 0)),
          pl.BlockSpec((h, h2), lambda i: (0, 0)),
          pl.BlockSpec((h, h2), lambda i: (0, 0)),
      ],
      out_specs=[
          pl.BlockSpec((block_rows, h2), lambda i: (i, 0)),
          pl.BlockSpec((block_rows, h2), lambda i: (i, 0)),
      ],
      out_shape=[
          jax.ShapeDtypeStruct((n, h2), jnp.float32),
          jax.ShapeDtypeStruct((n, h2), jnp.float32),
      ],
  )(acc, cnt, r1, b1, w2l, w2r)


def _combine_final(acc, cnt, r2, b2, block_rows):
  _, n, h = acc.shape

  def body(acc_ref, cnt_ref, r2_ref, b2_ref, o_ref):
    a = acc_ref[0] + acc_ref[1]
    cn = jnp.maximum(cnt_ref[0, :, 0:1] + cnt_ref[1, :, 0:1], 1.0)
    o_ref[...] = jnp.maximum(a / cn + r2_ref[...] + b2_ref[...], 0.0)

  return pl.pallas_call(
      body,
      grid=(n // block_rows,),
      in_specs=[
          pl.BlockSpec((NC, block_rows, h), lambda i: (0, i, 0)),
          pl.BlockSpec((NC, block_rows, CW), lambda i: (0, i, 0)),
          pl.BlockSpec((block_rows, h), lambda i: (i, 0)),
          pl.BlockSpec((1, h), lambda i: (0, 0)),
      ],
      out_specs=pl.BlockSpec((block_rows, h), lambda i: (i, 0)),
      out_shape=jax.ShapeDtypeStruct((n, h), jnp.float32),
  )(acc, cnt, r2, b2)


def kernel(x, edge_index, W1_l, W1_r, b1, W2_l, W2_r, b2):
  n, d = x.shape
  h = W1_l.shape[1]
  e = edge_index.shape[1]
  n_chunks = e // (NW * CHUNK)
  block_rows = 1000

  step = ((n // NS) // 8) * 8
  stripe_rows = n - (NS - 1) * step
  src2 = edge_index[0].reshape(NW, n_chunks, CHUNK)
  dst2 = edge_index[1].reshape(NW, n_chunks, CHUNK)
  z32 = jnp.zeros((stripe_rows, h), jnp.float32)
  z16 = jnp.zeros((stripe_rows, CW), jnp.float32)
  ones = jnp.ones((CHUNK, CW), jnp.float32)

  sc_pass1 = _make_sc_edge_pass(n, h, n_chunks, do_cnt=True)
  sc_pass2 = _make_sc_edge_pass(n, h, n_chunks, do_cnt=False)

  p1, r1 = _mm_dual(x, W1_l, W1_r, block_rows)
  acc1, cnt = sc_pass1(p1, z32, z16, ones, src2, dst2)
  p2, r2 = _combine_mm(acc1, cnt, r1, b1.reshape(1, h), W2_l, W2_r, block_rows)
  acc2 = sc_pass2(p2, z32, src2, dst2)
  return _combine_final(acc2, cnt, r2, b2.reshape(1, h), block_rows)


# speedup vs baseline: 17.9829x; 1.0278x over previous
import functools

import jax
import jax.numpy as jnp
from jax import lax
from jax.experimental import pallas as pl
from jax.experimental.pallas import tpu as pltpu
from jax.experimental.pallas import tpu_sc as plsc

NC = 2
NS = 16
NW = NC * NS
CHUNK = 100
CW = 16
NBUF = 4

_MESH = plsc.VectorSubcoreMesh(
    core_axis_name="c", subcore_axis_name="s", num_cores=NC, num_subcores=NS
)


def _make_sc_edge_pass(n, h, n_chunks, do_cnt):
  step = ((n // NS) // 8) * 8
  stripe_rows = n - (NS - 1) * step
  out_type = [jax.ShapeDtypeStruct((NC, n, h), jnp.float32)]
  scratch = [
      pltpu.VMEM((n_chunks, CHUNK), jnp.int32),
      pltpu.VMEM((n_chunks, CHUNK), jnp.int32),
      [pltpu.VMEM((CHUNK, h), jnp.float32) for _ in range(NBUF)],
      pltpu.VMEM_SHARED((n, h), jnp.float32),
      pltpu.VMEM_SHARED((n, h), jnp.float32),
      [pltpu.SemaphoreType.DMA for _ in range(NBUF)],
  ]
  if do_cnt:
    out_type.append(jax.ShapeDtypeStruct((NC, n, CW), jnp.float32))
    scratch += [
        pltpu.VMEM((CHUNK, CW), jnp.float32),
        pltpu.VMEM_SHARED((n, CW), jnp.float32),
        pltpu.SemaphoreType.DMA,
    ]

  def body(*refs):
    if do_cnt:
      (p_hbm, z32, z16, ones_hbm, src_hbm, dst_hbm,
       acc_out, cnt_out, srcv, dstv, rows, acc_sp, tbl_sp,
       sems, onesv, cnt_sp, csem) = refs
    else:
      (p_hbm, z32, src_hbm, dst_hbm,
       acc_out, srcv, dstv, rows, acc_sp, tbl_sp, sems) = refs
    c = lax.axis_index("c")
    s = lax.axis_index("s")
    wid = s * NC + c
    r0 = pl.multiple_of(s * step, 8)
    stripe = pl.ds(r0, stripe_rows)
    pltpu.sync_copy(p_hbm.at[stripe], tbl_sp.at[stripe])
    pltpu.sync_copy(z32, acc_sp.at[stripe])
    if do_cnt:
      pltpu.sync_copy(z16, cnt_sp.at[stripe])
      pltpu.sync_copy(ones_hbm, onesv)
    pltpu.sync_copy(src_hbm.at[wid], srcv)
    pltpu.sync_copy(dst_hbm.at[wid], dstv)
    plsc.subcore_barrier()

    for k in range(NBUF - 1):
      pltpu.async_copy(tbl_sp.at[srcv.at[k]], rows[k], sems[k])

    def edge_step(i, carry):
      for k in range(NBUF):
        j = NBUF * i + k
        nxt = (k + NBUF - 1) % NBUF

        @pl.when(j + NBUF - 1 < n_chunks)
        def _():
          pltpu.async_copy(tbl_sp.at[srcv.at[j + NBUF - 1]], rows[nxt],
                           sems[nxt])

        pltpu.make_async_copy(tbl_sp.at[srcv.at[j]], rows[k], sems[k]).wait()
        pltpu.sync_copy(rows[k], acc_sp.at[dstv.at[j]], add=True)
        if do_cnt:
          pltpu.async_copy(onesv, cnt_sp.at[dstv.at[j]], csem, add=True)
      return carry

    lax.fori_loop(0, n_chunks // NBUF, edge_step, 0)
    if do_cnt:
      def cnt_drain(j, carry):
        pltpu.make_async_copy(onesv, cnt_sp.at[dstv.at[0]], csem).wait()
        return carry

      lax.fori_loop(0, n_chunks, cnt_drain, 0)
    plsc.subcore_barrier()
    pltpu.sync_copy(acc_sp.at[stripe], acc_out.at[c, stripe])
    if do_cnt:
      pltpu.sync_copy(cnt_sp.at[stripe], cnt_out.at[c, stripe])

  return pl.kernel(body, out_type=out_type if do_cnt else out_type[0],
                   mesh=_MESH, scratch_types=scratch,
                   compiler_params=pltpu.CompilerParams(
                       use_tc_tiling_on_sc=False))


def _mm_dual(x, wl, wr, block_rows):
  n, d = x.shape
  h = wl.shape[1]

  def body(x_ref, wl_ref, wr_ref, p_ref, r_ref):
    xb = x_ref[...]
    p_ref[...] = jnp.dot(xb, wl_ref[...], preferred_element_type=jnp.float32)
    r_ref[...] = jnp.dot(xb, wr_ref[...], preferred_element_type=jnp.float32)

  return pl.pallas_call(
      body,
      grid=(n // block_rows,),
      in_specs=[
          pl.BlockSpec((block_rows, d), lambda i: (i, 0)),
          pl.BlockSpec((d, h), lambda i: (0, 0)),
          pl.BlockSpec((d, h), lambda i: (0, 0)),
      ],
      out_specs=[
          pl.BlockSpec((block_rows, h), lambda i: (i, 0)),
          pl.BlockSpec((block_rows, h), lambda i: (i, 0)),
      ],
      out_shape=[
          jax.ShapeDtypeStruct((n, h), jnp.float32),
          jax.ShapeDtypeStruct((n, h), jnp.float32),
      ],
  )(x, wl, wr)


def _combine_mm(acc, cnt, r1, b1, w2l, w2r, block_rows):
  _, n, h = acc.shape
  h2 = w2l.shape[1]

  def body(acc_ref, cnt_ref, r1_ref, b1_ref, w2l_ref, w2r_ref, p_ref, r_ref):
    a = acc_ref[0] + acc_ref[1]
    cn = jnp.maximum(cnt_ref[0, :, 0:1] + cnt_ref[1, :, 0:1], 1.0)
    hb = jnp.maximum(a / cn + r1_ref[...] + b1_ref[...], 0.0)
    p_ref[...] = jnp.dot(hb, w2l_ref[...], preferred_element_type=jnp.float32)
    r_ref[...] = jnp.dot(hb, w2r_ref[...], preferred_element_type=jnp.float32)

  return pl.pallas_call(
      body,
      grid=(n // block_rows,),
      in_specs=[
          pl.BlockSpec((NC, block_rows, h), lambda i: (0, i, 0)),
          pl.BlockSpec((NC, block_rows, CW), lambda i: (0, i, 0)),
          pl.BlockSpec((block_rows, h), lambda i: (i, 0)),
          pl.BlockSpec((1, h), lambda i: (0, 0)),
          pl.BlockSpec((h, h2), lambda i: (0, 0)),
          pl.BlockSpec((h, h2), lambda i: (0, 0)),
      ],
      out_specs=[
          pl.BlockSpec((block_rows, h2), lambda i: (i, 0)),
          pl.BlockSpec((block_rows, h2), lambda i: (i, 0)),
      ],
      out_shape=[
          jax.ShapeDtypeStruct((n, h2), jnp.float32),
          jax.ShapeDtypeStruct((n, h2), jnp.float32),
      ],
  )(acc, cnt, r1, b1, w2l, w2r)


def _combine_final(acc, cnt, r2, b2, block_rows):
  _, n, h = acc.shape

  def body(acc_ref, cnt_ref, r2_ref, b2_ref, o_ref):
    a = acc_ref[0] + acc_ref[1]
    cn = jnp.maximum(cnt_ref[0, :, 0:1] + cnt_ref[1, :, 0:1], 1.0)
    o_ref[...] = jnp.maximum(a / cn + r2_ref[...] + b2_ref[...], 0.0)

  return pl.pallas_call(
      body,
      grid=(n // block_rows,),
      in_specs=[
          pl.BlockSpec((NC, block_rows, h), lambda i: (0, i, 0)),
          pl.BlockSpec((NC, block_rows, CW), lambda i: (0, i, 0)),
          pl.BlockSpec((block_rows, h), lambda i: (i, 0)),
          pl.BlockSpec((1, h), lambda i: (0, 0)),
      ],
      out_specs=pl.BlockSpec((block_rows, h), lambda i: (i, 0)),
      out_shape=jax.ShapeDtypeStruct((n, h), jnp.float32),
  )(acc, cnt, r2, b2)


def kernel(x, edge_index, W1_l, W1_r, b1, W2_l, W2_r, b2):
  n, d = x.shape
  h = W1_l.shape[1]
  e = edge_index.shape[1]
  n_chunks = e // (NW * CHUNK)
  block_rows = 1000

  step = ((n // NS) // 8) * 8
  stripe_rows = n - (NS - 1) * step
  src2 = edge_index[0].reshape(NW, n_chunks, CHUNK)
  dst2 = edge_index[1].reshape(NW, n_chunks, CHUNK)
  z32 = jnp.zeros((stripe_rows, h), jnp.float32)
  z16 = jnp.zeros((stripe_rows, CW), jnp.float32)
  ones = jnp.ones((CHUNK, CW), jnp.float32)

  sc_pass1 = _make_sc_edge_pass(n, h, n_chunks, do_cnt=True)
  sc_pass2 = _make_sc_edge_pass(n, h, n_chunks, do_cnt=False)

  p1, r1 = _mm_dual(x, W1_l, W1_r, block_rows)
  acc1, cnt = sc_pass1(p1, z32, z16, ones, src2, dst2)
  p2, r2 = _combine_mm(acc1, cnt, r1, b1.reshape(1, h), W2_l, W2_r, block_rows)
  acc2 = sc_pass2(p2, z32, src2, dst2)
  return _combine_final(acc2, cnt, r2, b2.reshape(1, h), block_rows)


# speedup vs baseline: 19.8780x; 1.1054x over previous
import functools

import jax
import jax.numpy as jnp
from jax import lax
from jax.experimental import pallas as pl
from jax.experimental.pallas import tpu as pltpu
from jax.experimental.pallas import tpu_sc as plsc

NC = 2
NS = 16
NW = NC * NS
CHUNK = 100
CW = 32
NBUF = 4
PK = 4

_MESH = plsc.VectorSubcoreMesh(
    core_axis_name="c", subcore_axis_name="s", num_cores=NC, num_subcores=NS
)


def _make_sc_edge_pass(n, h, n_chunks, do_cnt):
  step = ((n // NS) // 8) * 8
  stripe_rows = n - (NS - 1) * step
  out_type = [jax.ShapeDtypeStruct((NC, n, h), jnp.float32)]
  scratch = [
      pltpu.VMEM((n_chunks, CHUNK), jnp.int32),
      pltpu.VMEM((n_chunks, CHUNK), jnp.int32),
      [pltpu.VMEM((CHUNK, h), jnp.float32) for _ in range(NBUF)],
      pltpu.VMEM_SHARED((n, h), jnp.float32),
      pltpu.VMEM_SHARED((n, h), jnp.float32),
      [pltpu.SemaphoreType.DMA for _ in range(NBUF)],
  ]
  if do_cnt:
    out_type.append(jax.ShapeDtypeStruct((NC, n, CW), jnp.float32))
    scratch += [
        pltpu.VMEM((CHUNK, CW), jnp.float32),
        pltpu.VMEM_SHARED((n, CW), jnp.float32),
        pltpu.SemaphoreType.DMA,
    ]

  def body(*refs):
    if do_cnt:
      (p_hbm, z32, z16, ones_hbm, src_hbm, dst_hbm,
       acc_out, cnt_out, srcv, dstv, rows, acc_sp, tbl_sp,
       sems, onesv, cnt_sp, csem) = refs
    else:
      (p_hbm, z32, src_hbm, dst_hbm,
       acc_out, srcv, dstv, rows, acc_sp, tbl_sp, sems) = refs
    c = lax.axis_index("c")
    s = lax.axis_index("s")
    wid = s * NC + c
    r0 = pl.multiple_of(s * step, 8)
    stripe = pl.ds(r0, stripe_rows)
    pltpu.sync_copy(p_hbm.at[stripe], tbl_sp.at[stripe])
    pltpu.sync_copy(z32, acc_sp.at[stripe])
    if do_cnt:
      pltpu.sync_copy(z16, cnt_sp.at[stripe])
      pltpu.sync_copy(ones_hbm, onesv)
    pltpu.sync_copy(src_hbm.at[wid], srcv)
    pltpu.sync_copy(dst_hbm.at[wid], dstv)
    plsc.subcore_barrier()

    for k in range(NBUF - 1):
      pltpu.async_copy(tbl_sp.at[srcv.at[k]], rows[k], sems[k])

    def edge_step(i, carry):
      for k in range(NBUF):
        j = NBUF * i + k
        nxt = (k + NBUF - 1) % NBUF

        @pl.when(j + NBUF - 1 < n_chunks)
        def _():
          pltpu.async_copy(tbl_sp.at[srcv.at[j + NBUF - 1]], rows[nxt],
                           sems[nxt])

        pltpu.make_async_copy(tbl_sp.at[srcv.at[j]], rows[k], sems[k]).wait()
        pltpu.sync_copy(rows[k], acc_sp.at[dstv.at[j]], add=True)
        if do_cnt:
          pltpu.async_copy(onesv, cnt_sp.at[dstv.at[j]], csem, add=True)
      return carry

    lax.fori_loop(0, n_chunks // NBUF, edge_step, 0)
    if do_cnt:
      def cnt_drain(j, carry):
        pltpu.make_async_copy(onesv, cnt_sp.at[dstv.at[0]], csem).wait()
        return carry

      lax.fori_loop(0, n_chunks, cnt_drain, 0)
    plsc.subcore_barrier()
    pltpu.sync_copy(acc_sp.at[stripe], acc_out.at[c, stripe])
    if do_cnt:
      pltpu.sync_copy(cnt_sp.at[stripe], cnt_out.at[c, stripe])

  return pl.kernel(body, out_type=out_type if do_cnt else out_type[0],
                   mesh=_MESH, scratch_types=scratch,
                   compiler_params=pltpu.CompilerParams(
                       use_tc_tiling_on_sc=False))


def _mm_dual_packed(x4, wl4, wr4, block_rows):
  r, d4 = x4.shape
  m = wl4.shape[1]

  def body(x_ref, wl_ref, wr_ref, p_ref, q_ref):
    xb = x_ref[...]
    p_ref[...] = jnp.dot(xb, wl_ref[...], preferred_element_type=jnp.float32)
    q_ref[...] = jnp.dot(xb, wr_ref[...], preferred_element_type=jnp.float32)

  return pl.pallas_call(
      body,
      grid=(pl.cdiv(r, block_rows),),
      in_specs=[
          pl.BlockSpec((block_rows, d4), lambda i: (i, 0)),
          pl.BlockSpec((d4, m), lambda i: (0, 0)),
          pl.BlockSpec((d4, m), lambda i: (0, 0)),
      ],
      out_specs=[
          pl.BlockSpec((block_rows, m), lambda i: (i, 0)),
          pl.BlockSpec((block_rows, m), lambda i: (i, 0)),
      ],
      out_shape=[
          jax.ShapeDtypeStruct((r, m), jnp.float32),
          jax.ShapeDtypeStruct((r, m), jnp.float32),
      ],
  )(x4, wl4, wr4)


def _combine_packed(acc, cnt, rpk, bpk, w2l4, w2r4, block_rows):
  _, r, m = acc.shape

  def body(acc_ref, cnt_ref, r_ref, b_ref, wl_ref, wr_ref, p_ref, q_ref):
    a = acc_ref[0] + acc_ref[1]
    cn = jnp.maximum(cnt_ref[0] + cnt_ref[1], 1.0)
    hb = jnp.maximum(a / cn + r_ref[...] + b_ref[...], 0.0)
    p_ref[...] = jnp.dot(hb, wl_ref[...], preferred_element_type=jnp.float32)
    q_ref[...] = jnp.dot(hb, wr_ref[...], preferred_element_type=jnp.float32)

  return pl.pallas_call(
      body,
      grid=(pl.cdiv(r, block_rows),),
      in_specs=[
          pl.BlockSpec((NC, block_rows, m), lambda i: (0, i, 0)),
          pl.BlockSpec((NC, block_rows, m), lambda i: (0, i, 0)),
          pl.BlockSpec((block_rows, m), lambda i: (i, 0)),
          pl.BlockSpec((1, m), lambda i: (0, 0)),
          pl.BlockSpec((m, m), lambda i: (0, 0)),
          pl.BlockSpec((m, m), lambda i: (0, 0)),
      ],
      out_specs=[
          pl.BlockSpec((block_rows, m), lambda i: (i, 0)),
          pl.BlockSpec((block_rows, m), lambda i: (i, 0)),
      ],
      out_shape=[
          jax.ShapeDtypeStruct((r, m), jnp.float32),
          jax.ShapeDtypeStruct((r, m), jnp.float32),
      ],
  )(acc, cnt, rpk, bpk, w2l4, w2r4)


def _final_packed(acc, cnt, rpk, bpk, block_rows):
  _, r, m = acc.shape

  def body(acc_ref, cnt_ref, r_ref, b_ref, o_ref):
    a = acc_ref[0] + acc_ref[1]
    cn = jnp.maximum(cnt_ref[0] + cnt_ref[1], 1.0)
    o_ref[...] = jnp.maximum(a / cn + r_ref[...] + b_ref[...], 0.0)

  return pl.pallas_call(
      body,
      grid=(pl.cdiv(r, block_rows),),
      in_specs=[
          pl.BlockSpec((NC, block_rows, m), lambda i: (0, i, 0)),
          pl.BlockSpec((NC, block_rows, m), lambda i: (0, i, 0)),
          pl.BlockSpec((block_rows, m), lambda i: (i, 0)),
          pl.BlockSpec((1, m), lambda i: (0, 0)),
      ],
      out_specs=pl.BlockSpec((block_rows, m), lambda i: (i, 0)),
      out_shape=jax.ShapeDtypeStruct((r, m), jnp.float32),
  )(acc, cnt, rpk, bpk)


def kernel(x, edge_index, W1_l, W1_r, b1, W2_l, W2_r, b2):
  n, d = x.shape
  h = W1_l.shape[1]
  e = edge_index.shape[1]
  n_chunks = e // (NW * CHUNK)
  block_rows = 256
  rpk = n // PK

  step = ((n // NS) // 8) * 8
  stripe_rows = n - (NS - 1) * step
  src2 = edge_index[0].reshape(NW, n_chunks, CHUNK)
  dst2 = edge_index[1].reshape(NW, n_chunks, CHUNK)
  z32 = jnp.zeros((stripe_rows, h), jnp.float32)
  ones = jnp.ones((CHUNK, CW), jnp.float32)

  eye = jnp.eye(PK, dtype=jnp.float32)
  w1l4 = jnp.kron(eye, W1_l)
  w1r4 = jnp.kron(eye, W1_r)
  w2l4 = jnp.kron(eye, W2_l)
  w2r4 = jnp.kron(eye, W2_r)
  b1pk = jnp.tile(b1, PK).reshape(1, PK * h)
  b2pk = jnp.tile(b2, PK).reshape(1, PK * h)

  sc_pass1 = _make_sc_edge_pass(n, h, n_chunks, do_cnt=True)
  sc_pass2 = _make_sc_edge_pass(n, h, n_chunks, do_cnt=False)

  x4 = x.reshape(rpk, PK * d)
  p1pk, r1pk = _mm_dual_packed(x4, w1l4, w1r4, block_rows)
  acc1, cnt = sc_pass1(p1pk.reshape(n, h), z32, z32, ones, src2, dst2)
  p2pk, r2pk = _combine_packed(
      acc1.reshape(NC, rpk, PK * h), cnt.reshape(NC, rpk, PK * h),
      r1pk, b1pk, w2l4, w2r4, block_rows)
  acc2 = sc_pass2(p2pk.reshape(n, h), z32, src2, dst2)
  outpk = _final_packed(
      acc2.reshape(NC, rpk, PK * h), cnt.reshape(NC, rpk, PK * h),
      r2pk, b2pk, block_rows)
  return outpk.reshape(n, h)
